# baseline (speedup 1.0000x reference)
import functools

import jax
import jax.numpy as jnp
from jax.experimental import pallas as pl
from jax.experimental.pallas import tpu as pltpu

H, E, FF, TOP_K = 1024, 8, 2048, 2
N = 2048
BT = 512
NB = N // BT


def _moe_kernel(x_ref, wr_ref, w1_ref, b1_ref, w2_ref, b2_ref,
                out_ref, aux_ref,
                i1_s, i2_s, wa_s, wb_s, cnt_s):
    e = pl.program_id(0)
    nb = pl.program_id(1)
    rows = pl.ds(nb * BT, BT)
    xb = x_ref[...]

    @pl.when(e == 0)
    def _router():
        lg = jnp.dot(xb, wr_ref[...], preferred_element_type=jnp.float32)
        ids = jax.lax.broadcasted_iota(jnp.int32, (BT, E), 1)
        m1 = jnp.max(lg, axis=1, keepdims=True)
        i1 = jnp.min(jnp.where(lg == m1, ids, E), axis=1, keepdims=True)
        masked = jnp.where(ids == i1, -jnp.inf, lg)
        m2 = jnp.max(masked, axis=1, keepdims=True)
        i2 = jnp.min(jnp.where(masked == m2, ids, E), axis=1, keepdims=True)
        r = jnp.exp(m2 - m1)
        wa = 1.0 / (1.0 + r)
        wb = 1.0 - wa
        i1_s[rows, :] = i1
        i2_s[rows, :] = i2
        wa_s[rows, :] = wa
        wb_s[rows, :] = wb
        cblk = (jnp.sum((ids == i1).astype(jnp.float32), axis=0, keepdims=True)
                + jnp.sum((ids == i2).astype(jnp.float32), axis=0, keepdims=True))

        @pl.when(nb == 0)
        def _():
            cnt_s[...] = cblk

        @pl.when(nb > 0)
        def _():
            cnt_s[...] = cnt_s[...] + cblk

    h = jnp.dot(xb.astype(jnp.bfloat16), w1_ref[0],
                preferred_element_type=jnp.float32)
    h = h + b1_ref[0]
    h = 0.5 * h * (1.0 + jax.lax.erf(h * 0.7071067811865476))
    y = jnp.dot(h.astype(jnp.bfloat16), w2_ref[0],
                preferred_element_type=jnp.float32) + b2_ref[0]

    w_col = (jnp.where(i1_s[rows, :] == e, wa_s[rows, :], 0.0)
             + jnp.where(i2_s[rows, :] == e, wb_s[rows, :], 0.0))
    contrib = y * w_col

    @pl.when(e == 0)
    def _():
        out_ref[rows, :] = contrib

    @pl.when(e > 0)
    def _():
        out_ref[rows, :] = out_ref[rows, :] + contrib

    @pl.when((e == E - 1) & (nb == NB - 1))
    def _aux():
        counts = cnt_s[...]
        load = counts / jnp.sum(counts)
        aux = 0.01 * jnp.sum(load * jnp.log(load + 1e-9), axis=1, keepdims=True)
        aux_ref[...] = aux


@jax.jit
def kernel(x, Wr, W1, b1, W2, b2):
    B, L, Hd = x.shape
    xf = x.reshape(-1, Hd)
    out, aux = pl.pallas_call(
        _moe_kernel,
        grid=(E, NB),
        in_specs=[
            pl.BlockSpec((BT, H), lambda e, nb: (nb, 0)),
            pl.BlockSpec((H, E), lambda e, nb: (0, 0)),
            pl.BlockSpec((1, H, FF), lambda e, nb: (e, 0, 0)),
            pl.BlockSpec((1, 1, FF), lambda e, nb: (e, 0, 0)),
            pl.BlockSpec((1, FF, H), lambda e, nb: (e, 0, 0)),
            pl.BlockSpec((1, 1, H), lambda e, nb: (e, 0, 0)),
        ],
        out_specs=[
            pl.BlockSpec((N, H), lambda e, nb: (0, 0)),
            pl.BlockSpec((1, 1), lambda e, nb: (0, 0)),
        ],
        out_shape=[
            jax.ShapeDtypeStruct((N, H), jnp.float32),
            jax.ShapeDtypeStruct((1, 1), jnp.float32),
        ],
        scratch_shapes=[
            pltpu.VMEM((N, 1), jnp.int32),
            pltpu.VMEM((N, 1), jnp.int32),
            pltpu.VMEM((N, 1), jnp.float32),
            pltpu.VMEM((N, 1), jnp.float32),
            pltpu.VMEM((1, E), jnp.float32),
        ],
    )(xf, Wr, W1.astype(jnp.bfloat16), b1.reshape(E, 1, FF),
      W2.astype(jnp.bfloat16), b2.reshape(E, 1, H))
    return out.reshape(B, L, Hd), aux.reshape(())


# speedup vs baseline: 1.2383x; 1.2383x over previous
import functools

import jax
import jax.numpy as jnp
from jax import lax
from jax.experimental import pallas as pl
from jax.experimental.pallas import tpu as pltpu
from jax.experimental.pallas import tpu_sc as plsc

H, E, FF = 1024, 8, 2048
N = 2048
W = 32
TPW = N // W
G = 8
BT = 256
P = 8192
NBLK = P // BT
STG = 192
NMETA = 64


def _router_kernel(x_ref, wr_ref, i1_ref, i2_ref, wa_ref, wb_ref, cnt_ref,
                   meta_ref, aux_ref):
    lg = jnp.dot(x_ref[...], wr_ref[...], preferred_element_type=jnp.float32)
    ids = lax.broadcasted_iota(jnp.int32, (N, E), 1)
    m1 = jnp.max(lg, axis=1, keepdims=True)
    i1 = jnp.min(jnp.where(lg == m1, ids, E), axis=1, keepdims=True)
    masked = jnp.where(ids == i1, -jnp.inf, lg)
    m2 = jnp.max(masked, axis=1, keepdims=True)
    i2 = jnp.min(jnp.where(masked == m2, ids, E), axis=1, keepdims=True)
    r = jnp.exp(m2 - m1)
    wa = 1.0 / (1.0 + r)
    i1_ref[...] = i1
    i2_ref[...] = i2
    wa_ref[...] = wa
    wb_ref[...] = 1.0 - wa
    ids16 = lax.broadcasted_iota(jnp.int32, (N, 16), 1)
    oh = (ids16 == i1).astype(jnp.float32) + (ids16 == i2).astype(jnp.float32)
    grp = lax.broadcasted_iota(jnp.int32, (W, N), 0)
    tokg = lax.broadcasted_iota(jnp.int32, (W, N), 1) // TPW
    sel = (grp == tokg).astype(jnp.float32)
    cnt = jnp.dot(sel, oh, preferred_element_type=jnp.float32)
    cnti = cnt.astype(jnp.int32)
    cnt_ref[...] = cnti
    rpad = (cnti + (G - 1)) & (-G)
    ptot = jnp.sum(rpad, axis=0, keepdims=True)
    region = (ptot + (BT - 1)) & (-BT)
    ii = lax.broadcasted_iota(jnp.int32, (16, 16), 0)
    jj = lax.broadcasted_iota(jnp.int32, (16, 16), 1)
    lt = (ii < jj).astype(jnp.float32)
    bend = (jnp.dot(region.astype(jnp.float32), lt,
                    preferred_element_type=jnp.float32)
            + region.astype(jnp.float32))
    bids = lax.broadcasted_iota(jnp.int32, (NMETA, 16), 0) * BT
    over = (bids.astype(jnp.float32) >= bend).astype(jnp.float32)
    colmask = lax.broadcasted_iota(jnp.int32, (NMETA, 16), 1) < E
    meta = jnp.sum(jnp.where(colmask, over, 0.0), axis=1, keepdims=True)
    meta_ref[...] = meta.astype(jnp.int32)
    counts = jnp.sum(oh[:, :E], axis=0, keepdims=True)
    load = counts / jnp.sum(counts)
    aux_ref[...] = 0.01 * jnp.sum(load * jnp.log(load + 1e-9), axis=1, keepdims=True)


def _router(xf, Wr):
    return pl.pallas_call(
        _router_kernel,
        grid=(1,),
        in_specs=[
            pl.BlockSpec((N, H), lambda i: (0, 0)),
            pl.BlockSpec((H, E), lambda i: (0, 0)),
        ],
        out_specs=[
            pl.BlockSpec((N, 1), lambda i: (0, 0)),
            pl.BlockSpec((N, 1), lambda i: (0, 0)),
            pl.BlockSpec((N, 1), lambda i: (0, 0)),
            pl.BlockSpec((N, 1), lambda i: (0, 0)),
            pl.BlockSpec((W, 16), lambda i: (0, 0)),
            pl.BlockSpec((NMETA, 1), lambda i: (0, 0)),
            pl.BlockSpec((1, 1), lambda i: (0, 0)),
        ],
        out_shape=[
            jax.ShapeDtypeStruct((N, 1), jnp.int32),
            jax.ShapeDtypeStruct((N, 1), jnp.int32),
            jax.ShapeDtypeStruct((N, 1), jnp.float32),
            jax.ShapeDtypeStruct((N, 1), jnp.float32),
            jax.ShapeDtypeStruct((W, 16), jnp.int32),
            jax.ShapeDtypeStruct((NMETA, 1), jnp.int32),
            jax.ShapeDtypeStruct((1, 1), jnp.float32),
        ],
    )(xf, Wr)


def _make_dispatch():
    mesh = plsc.VectorSubcoreMesh(core_axis_name="c", subcore_axis_name="s")

    @functools.partial(
        pl.kernel,
        mesh=mesh,
        out_type=[
            jax.ShapeDtypeStruct((P, H), jnp.float32),
            jax.ShapeDtypeStruct((N,), jnp.int32),
            jax.ShapeDtypeStruct((N,), jnp.int32),
        ],
        scratch_types=[
            pltpu.VMEM((TPW,), jnp.int32),
            pltpu.VMEM((TPW,), jnp.int32),
            pltpu.VMEM((W * 16,), jnp.int32),
            pltpu.VMEM((TPW,), jnp.int32),
            pltpu.VMEM((TPW,), jnp.int32),
            pltpu.VMEM((TPW, H), jnp.float32),
            pltpu.SemaphoreType.DMA,
        ],
    )
    def dispatch(i1_hbm, i2_hbm, cnt_hbm, x_hbm,
                 xg_hbm, pos1_hbm, pos2_hbm,
                 i1_v, i2_v, cntg_v, pos1_v, pos2_v, xrows_v, sem):
        wid = lax.axis_index("s") * 2 + lax.axis_index("c")
        lane = lax.broadcasted_iota(jnp.int32, (16,), 0)
        t0 = pl.multiple_of(wid * TPW, TPW)
        pltpu.sync_copy(i1_hbm.at[pl.ds(t0, TPW)], i1_v)
        pltpu.sync_copy(i2_hbm.at[pl.ds(t0, TPW)], i2_v)
        pltpu.sync_copy(cnt_hbm, cntg_v)
        pltpu.sync_copy(x_hbm.at[pl.ds(t0, TPW)], xrows_v)

        zeros16 = jnp.zeros((16,), jnp.int32)
        totpad = zeros16
        mypre = zeros16
        myrow = zeros16
        for w in range(W):
            row = cntg_v[pl.ds(w * 16, 16)]
            rpad = (row + (G - 1)) & (-G)
            totpad = totpad + rpad
            mypre = mypre + rpad * (w < wid).astype(jnp.int32)
            myrow = myrow + row * (w == wid).astype(jnp.int32)
        region = (totpad + (BT - 1)) & (-BT)
        mypad = (myrow + (G - 1)) & (-G)

        starts = []
        bacc = jnp.int32(0)
        for e in range(E):
            starts.append(bacc + mypre[e])
            bacc = bacc + region[e]

        ks = [jnp.int32(0)] * E

        def assign(v, ks):
            rank = jnp.zeros((16,), jnp.int32)
            hist = jnp.zeros((16,), jnp.int32)
            for j in range(16):
                vj = v[j]
                rank = rank + jnp.where((v == vj) & (lane > j), 1, 0)
                hist = hist + jnp.where(lane == vj, 1, 0)
            slots = jnp.zeros((16,), jnp.int32)
            nks = []
            for e in range(E):
                m = v == e
                slots = jnp.where(m, starts[e] + ks[e] + rank, slots)
                nks.append(ks[e] + hist[e])
            return slots, nks

        for j in range(TPW // 16):
            v = i1_v[pl.ds(j * 16, 16)]
            slots, ks = assign(v, ks)
            pos1_v[pl.ds(j * 16, 16)] = slots
            pltpu.async_copy(
                xrows_v.at[pl.ds(j * 16, 16)], xg_hbm.at[slots], sem
            ).wait()
        for j in range(TPW // 16):
            v = i2_v[pl.ds(j * 16, 16)]
            slots, ks = assign(v, ks)
            pos2_v[pl.ds(j * 16, 16)] = slots
            pltpu.async_copy(
                xrows_v.at[pl.ds(j * 16, 16)], xg_hbm.at[slots], sem
            ).wait()

        pltpu.sync_copy(pos1_v, pos1_hbm.at[pl.ds(t0, TPW)])
        pltpu.sync_copy(pos2_v, pos2_hbm.at[pl.ds(t0, TPW)])

    return dispatch


def _ffn_kernel(meta_ref, xg_ref, w1_ref, b1_ref, w2_ref, b2_ref, yd_ref):
    b = pl.program_id(0)
    e = meta_ref[b]

    @pl.when(e < E)
    def _():
        h = jnp.dot(xg_ref[...], w1_ref[0], preferred_element_type=jnp.float32)
        h = h + b1_ref[0]
        h = 0.5 * h * (1.0 + lax.erf(h * 0.7071067811865476))
        y = jnp.dot(h, w2_ref[0], preferred_element_type=jnp.float32) + b2_ref[0]
        yd_ref[...] = y


def _ffn(meta, xg, W1b, b1r, W2b, b2r):
    def clamp(m, b):
        return jnp.where(m[b] < E, m[b], 0)

    grid_spec = pltpu.PrefetchScalarGridSpec(
        num_scalar_prefetch=1,
        grid=(NBLK,),
        in_specs=[
            pl.BlockSpec((BT, H), lambda b, m: (b, 0)),
            pl.BlockSpec((1, H, FF), lambda b, m: (clamp(m, b), 0, 0)),
            pl.BlockSpec((1, 1, FF), lambda b, m: (clamp(m, b), 0, 0)),
            pl.BlockSpec((1, FF, H), lambda b, m: (clamp(m, b), 0, 0)),
            pl.BlockSpec((1, 1, H), lambda b, m: (clamp(m, b), 0, 0)),
        ],
        out_specs=pl.BlockSpec((BT, H), lambda b, m: (b, 0)),
    )
    return pl.pallas_call(
        _ffn_kernel,
        grid_spec=grid_spec,
        out_shape=jax.ShapeDtypeStruct((P, H), jnp.float32),
    )(meta, xg, W1b, b1r, W2b, b2r)


def _make_combine():
    mesh = plsc.VectorSubcoreMesh(core_axis_name="c", subcore_axis_name="s")
    HALF = 32

    @functools.partial(
        pl.kernel,
        mesh=mesh,
        out_type=jax.ShapeDtypeStruct((N, H), jnp.float32),
        scratch_types=[
            pltpu.VMEM((TPW,), jnp.int32),
            pltpu.VMEM((TPW,), jnp.int32),
            pltpu.VMEM((TPW,), jnp.float32),
            pltpu.VMEM((TPW,), jnp.float32),
            pltpu.VMEM((HALF, H), jnp.float32),
            pltpu.VMEM((HALF, H), jnp.float32),
            pltpu.SemaphoreType.DMA,
            pltpu.SemaphoreType.DMA,
        ],
    )
    def combine(yd_hbm, pos1_hbm, pos2_hbm, wa_hbm, wb_hbm, out_hbm,
                p1_v, p2_v, wa_v, wb_v, g1_v, g2_v, sem1, sem2):
        wid = lax.axis_index("s") * 2 + lax.axis_index("c")
        t0 = pl.multiple_of(wid * TPW, TPW)
        pltpu.sync_copy(pos1_hbm.at[pl.ds(t0, TPW)], p1_v)
        pltpu.sync_copy(pos2_hbm.at[pl.ds(t0, TPW)], p2_v)
        pltpu.sync_copy(wa_hbm.at[pl.ds(t0, TPW)], wa_v)
        pltpu.sync_copy(wb_hbm.at[pl.ds(t0, TPW)], wb_v)
        for hh in range(TPW // HALF):
            c1 = pltpu.async_copy(
                yd_hbm.at[p1_v.at[pl.ds(hh * HALF, HALF)]], g1_v, sem1)
            c2 = pltpu.async_copy(
                yd_hbm.at[p2_v.at[pl.ds(hh * HALF, HALF)]], g2_v, sem2)
            c1.wait()
            c2.wait()
            for t in range(HALF):
                tok = hh * HALF + t
                wav = wa_v[pl.ds((tok // 16) * 16, 16)]
                wbv = wb_v[pl.ds((tok // 16) * 16, 16)]
                a = wav[tok % 16]
                bsc = wbv[tok % 16]

                def body(i, _, t=t, a=a, bsc=bsc):
                    r1 = g1_v[t, pl.ds(i * 16, 16)]
                    r2 = g2_v[t, pl.ds(i * 16, 16)]
                    g1_v[t, pl.ds(i * 16, 16)] = a * r1 + bsc * r2
                    return 0

                lax.fori_loop(0, H // 16, body, 0)
            pltpu.sync_copy(g1_v, out_hbm.at[pl.ds(t0 + hh * HALF, HALF)])

    return combine


@jax.jit
def kernel(x, Wr, W1, b1, W2, b2):
    B, L, Hd = x.shape
    xf = x.reshape(-1, Hd)
    b1r = b1.reshape(E, 1, FF)
    b2r = b2.reshape(E, 1, H)

    i1, i2, wa, wb, cnt, meta, aux = _router(xf, Wr)
    i1f, i2f = i1.reshape(N), i2.reshape(N)
    waf, wbf = wa.reshape(N), wb.reshape(N)

    xg, pos1, pos2 = _make_dispatch()(i1f, i2f, cnt.reshape(-1), xf)
    yd = _ffn(meta.reshape(NMETA), xg, W1, b1r, W2, b2r)
    out = _make_combine()(yd, pos1, pos2, waf, wbf)
    return out.reshape(B, L, Hd), aux.reshape(())


# speedup vs baseline: 1.2532x; 1.0120x over previous
import functools

import jax
import jax.numpy as jnp
from jax import lax
from jax.experimental import pallas as pl
from jax.experimental.pallas import tpu as pltpu
from jax.experimental.pallas import tpu_sc as plsc

H, E, FF = 1024, 8, 2048
N = 2048
W = 32
TPW = N // W
G = 8
BT = 256
P = 8192
NBLK = P // BT
STG = 192
NMETA = 64


def _router_kernel(x_ref, wr_ref, i1_ref, i2_ref, wa_ref, wb_ref, cnt_ref,
                   meta_ref, aux_ref):
    lg = jnp.dot(x_ref[...], wr_ref[...], preferred_element_type=jnp.float32)
    ids = lax.broadcasted_iota(jnp.int32, (N, E), 1)
    m1 = jnp.max(lg, axis=1, keepdims=True)
    i1 = jnp.min(jnp.where(lg == m1, ids, E), axis=1, keepdims=True)
    masked = jnp.where(ids == i1, -jnp.inf, lg)
    m2 = jnp.max(masked, axis=1, keepdims=True)
    i2 = jnp.min(jnp.where(masked == m2, ids, E), axis=1, keepdims=True)
    r = jnp.exp(m2 - m1)
    wa = 1.0 / (1.0 + r)
    i1_ref[...] = i1
    i2_ref[...] = i2
    wa_ref[...] = wa
    wb_ref[...] = 1.0 - wa
    ids16 = lax.broadcasted_iota(jnp.int32, (N, 16), 1)
    oh = (ids16 == i1).astype(jnp.float32) + (ids16 == i2).astype(jnp.float32)
    grp = lax.broadcasted_iota(jnp.int32, (W, N), 0)
    tokg = lax.broadcasted_iota(jnp.int32, (W, N), 1) // TPW
    sel = (grp == tokg).astype(jnp.float32)
    cnt = jnp.dot(sel, oh, preferred_element_type=jnp.float32)
    cnti = cnt.astype(jnp.int32)
    cnt_ref[...] = cnti
    rpad = (cnti + (G - 1)) & (-G)
    ptot = jnp.sum(rpad, axis=0, keepdims=True)
    region = (ptot + (BT - 1)) & (-BT)
    ii = lax.broadcasted_iota(jnp.int32, (16, 16), 0)
    jj = lax.broadcasted_iota(jnp.int32, (16, 16), 1)
    lt = (ii < jj).astype(jnp.float32)
    bend = (jnp.dot(region.astype(jnp.float32), lt,
                    preferred_element_type=jnp.float32)
            + region.astype(jnp.float32))
    bids = lax.broadcasted_iota(jnp.int32, (NMETA, 16), 0) * BT
    over = (bids.astype(jnp.float32) >= bend).astype(jnp.float32)
    colmask = lax.broadcasted_iota(jnp.int32, (NMETA, 16), 1) < E
    meta = jnp.sum(jnp.where(colmask, over, 0.0), axis=1, keepdims=True)
    meta_ref[...] = meta.astype(jnp.int32)
    counts = jnp.sum(oh[:, :E], axis=0, keepdims=True)
    load = counts / jnp.sum(counts)
    aux_ref[...] = 0.01 * jnp.sum(load * jnp.log(load + 1e-9), axis=1, keepdims=True)


def _router(xf, Wr):
    return pl.pallas_call(
        _router_kernel,
        grid=(1,),
        in_specs=[
            pl.BlockSpec((N, H), lambda i: (0, 0)),
            pl.BlockSpec((H, E), lambda i: (0, 0)),
        ],
        out_specs=[
            pl.BlockSpec((N, 1), lambda i: (0, 0)),
            pl.BlockSpec((N, 1), lambda i: (0, 0)),
            pl.BlockSpec((N, 1), lambda i: (0, 0)),
            pl.BlockSpec((N, 1), lambda i: (0, 0)),
            pl.BlockSpec((W, 16), lambda i: (0, 0)),
            pl.BlockSpec((NMETA, 1), lambda i: (0, 0)),
            pl.BlockSpec((1, 1), lambda i: (0, 0)),
        ],
        out_shape=[
            jax.ShapeDtypeStruct((N, 1), jnp.int32),
            jax.ShapeDtypeStruct((N, 1), jnp.int32),
            jax.ShapeDtypeStruct((N, 1), jnp.float32),
            jax.ShapeDtypeStruct((N, 1), jnp.float32),
            jax.ShapeDtypeStruct((W, 16), jnp.int32),
            jax.ShapeDtypeStruct((NMETA, 1), jnp.int32),
            jax.ShapeDtypeStruct((1, 1), jnp.float32),
        ],
    )(xf, Wr)


def _make_dispatch():
    mesh = plsc.VectorSubcoreMesh(core_axis_name="c", subcore_axis_name="s")

    @functools.partial(
        pl.kernel,
        mesh=mesh,
        out_type=[
            jax.ShapeDtypeStruct((P, H), jnp.float32),
            jax.ShapeDtypeStruct((N,), jnp.int32),
            jax.ShapeDtypeStruct((N,), jnp.int32),
        ],
        scratch_types=[
            pltpu.VMEM((TPW,), jnp.int32),
            pltpu.VMEM((TPW,), jnp.int32),
            pltpu.VMEM((W * 16,), jnp.int32),
            pltpu.VMEM((TPW,), jnp.int32),
            pltpu.VMEM((TPW,), jnp.int32),
            pltpu.VMEM((TPW, H), jnp.float32),
            pltpu.SemaphoreType.DMA,
            pltpu.SemaphoreType.DMA,
        ],
    )
    def dispatch(i1_hbm, i2_hbm, cnt_hbm, x_hbm,
                 xg_hbm, pos1_hbm, pos2_hbm,
                 i1_v, i2_v, cntg_v, pos1_v, pos2_v, xrows_v, sem, sem2):
        wid = lax.axis_index("s") * 2 + lax.axis_index("c")
        lane = lax.broadcasted_iota(jnp.int32, (16,), 0)
        t0 = pl.multiple_of(wid * TPW, TPW)
        cx = pltpu.async_copy(x_hbm.at[pl.ds(t0, TPW)], xrows_v, sem2)
        ci1 = pltpu.async_copy(i1_hbm.at[pl.ds(t0, TPW)], i1_v, sem2)
        ci2 = pltpu.async_copy(i2_hbm.at[pl.ds(t0, TPW)], i2_v, sem2)
        pltpu.sync_copy(cnt_hbm, cntg_v)

        zeros16 = jnp.zeros((16,), jnp.int32)
        totpad = zeros16
        mypre = zeros16
        myrow = zeros16
        for w in range(W):
            row = cntg_v[pl.ds(w * 16, 16)]
            rpad = (row + (G - 1)) & (-G)
            totpad = totpad + rpad
            mypre = mypre + rpad * (w < wid).astype(jnp.int32)
            myrow = myrow + row * (w == wid).astype(jnp.int32)
        region = (totpad + (BT - 1)) & (-BT)
        mypad = (myrow + (G - 1)) & (-G)

        starts = []
        bacc = jnp.int32(0)
        for e in range(E):
            starts.append(bacc + mypre[e])
            bacc = bacc + region[e]

        ks = [jnp.int32(0)] * E

        def assign(v, ks):
            rank = jnp.zeros((16,), jnp.int32)
            hist = jnp.zeros((16,), jnp.int32)
            for j in range(16):
                vj = v[j]
                rank = rank + jnp.where((v == vj) & (lane > j), 1, 0)
                hist = hist + jnp.where(lane == vj, 1, 0)
            slots = jnp.zeros((16,), jnp.int32)
            nks = []
            for e in range(E):
                m = v == e
                slots = jnp.where(m, starts[e] + ks[e] + rank, slots)
                nks.append(ks[e] + hist[e])
            return slots, nks

        cx.wait()
        ci1.wait()
        ci2.wait()
        cps = []
        for j in range(TPW // 16):
            v = i1_v[pl.ds(j * 16, 16)]
            slots, ks = assign(v, ks)
            pos1_v[pl.ds(j * 16, 16)] = slots
            cps.append(pltpu.async_copy(
                xrows_v.at[pl.ds(j * 16, 16)], xg_hbm.at[slots], sem))
        for j in range(TPW // 16):
            v = i2_v[pl.ds(j * 16, 16)]
            slots, ks = assign(v, ks)
            pos2_v[pl.ds(j * 16, 16)] = slots
            cps.append(pltpu.async_copy(
                xrows_v.at[pl.ds(j * 16, 16)], xg_hbm.at[slots], sem))
        for c in cps:
            c.wait()

        pltpu.sync_copy(pos1_v, pos1_hbm.at[pl.ds(t0, TPW)])
        pltpu.sync_copy(pos2_v, pos2_hbm.at[pl.ds(t0, TPW)])

    return dispatch


def _ffn_kernel(meta_ref, xg_ref, w1_ref, b1_ref, w2_ref, b2_ref, yd_ref):
    b = pl.program_id(0)
    e = meta_ref[b]

    @pl.when(e < E)
    def _():
        h = jnp.dot(xg_ref[...], w1_ref[0], preferred_element_type=jnp.float32)
        h = h + b1_ref[0]
        h = 0.5 * h * (1.0 + lax.erf(h * 0.7071067811865476))
        y = jnp.dot(h, w2_ref[0], preferred_element_type=jnp.float32) + b2_ref[0]
        yd_ref[...] = y


def _ffn(meta, xg, W1b, b1r, W2b, b2r):
    def clamp(m, b):
        return jnp.where(m[b] < E, m[b], 0)

    grid_spec = pltpu.PrefetchScalarGridSpec(
        num_scalar_prefetch=1,
        grid=(NBLK,),
        in_specs=[
            pl.BlockSpec((BT, H), lambda b, m: (b, 0)),
            pl.BlockSpec((1, H, FF), lambda b, m: (clamp(m, b), 0, 0)),
            pl.BlockSpec((1, 1, FF), lambda b, m: (clamp(m, b), 0, 0)),
            pl.BlockSpec((1, FF, H), lambda b, m: (clamp(m, b), 0, 0)),
            pl.BlockSpec((1, 1, H), lambda b, m: (clamp(m, b), 0, 0)),
        ],
        out_specs=pl.BlockSpec((BT, H), lambda b, m: (b, 0)),
    )
    return pl.pallas_call(
        _ffn_kernel,
        grid_spec=grid_spec,
        out_shape=jax.ShapeDtypeStruct((P, H), jnp.float32),
    )(meta, xg, W1b, b1r, W2b, b2r)


def _make_combine():
    mesh = plsc.VectorSubcoreMesh(core_axis_name="c", subcore_axis_name="s")
    HALF = 32

    @functools.partial(
        pl.kernel,
        mesh=mesh,
        out_type=jax.ShapeDtypeStruct((N, H), jnp.float32),
        scratch_types=[
            pltpu.VMEM((TPW,), jnp.int32),
            pltpu.VMEM((TPW,), jnp.int32),
            pltpu.VMEM((TPW,), jnp.float32),
            pltpu.VMEM((TPW,), jnp.float32),
            pltpu.VMEM((HALF, H), jnp.float32),
            pltpu.VMEM((HALF, H), jnp.float32),
            pltpu.SemaphoreType.DMA,
            pltpu.SemaphoreType.DMA,
        ],
    )
    def combine(yd_hbm, pos1_hbm, pos2_hbm, wa_hbm, wb_hbm, out_hbm,
                p1_v, p2_v, wa_v, wb_v, g1_v, g2_v, sem1, sem2):
        wid = lax.axis_index("s") * 2 + lax.axis_index("c")
        t0 = pl.multiple_of(wid * TPW, TPW)
        pltpu.sync_copy(pos1_hbm.at[pl.ds(t0, TPW)], p1_v)
        pltpu.sync_copy(pos2_hbm.at[pl.ds(t0, TPW)], p2_v)
        pltpu.sync_copy(wa_hbm.at[pl.ds(t0, TPW)], wa_v)
        pltpu.sync_copy(wb_hbm.at[pl.ds(t0, TPW)], wb_v)
        for hh in range(TPW // HALF):
            c1 = pltpu.async_copy(
                yd_hbm.at[p1_v.at[pl.ds(hh * HALF, HALF)]], g1_v, sem1)
            c2 = pltpu.async_copy(
                yd_hbm.at[p2_v.at[pl.ds(hh * HALF, HALF)]], g2_v, sem2)
            c1.wait()
            c2.wait()
            for t in range(HALF):
                tok = hh * HALF + t
                wav = wa_v[pl.ds((tok // 16) * 16, 16)]
                wbv = wb_v[pl.ds((tok // 16) * 16, 16)]
                a = wav[tok % 16]
                bsc = wbv[tok % 16]

                def body(i, _, t=t, a=a, bsc=bsc):
                    r1 = g1_v[t, pl.ds(i * 16, 16)]
                    r2 = g2_v[t, pl.ds(i * 16, 16)]
                    g1_v[t, pl.ds(i * 16, 16)] = a * r1 + bsc * r2
                    return 0

                lax.fori_loop(0, H // 16, body, 0)
            pltpu.sync_copy(g1_v, out_hbm.at[pl.ds(t0 + hh * HALF, HALF)])

    return combine


@jax.jit
def kernel(x, Wr, W1, b1, W2, b2):
    B, L, Hd = x.shape
    xf = x.reshape(-1, Hd)
    b1r = b1.reshape(E, 1, FF)
    b2r = b2.reshape(E, 1, H)

    i1, i2, wa, wb, cnt, meta, aux = _router(xf, Wr)
    i1f, i2f = i1.reshape(N), i2.reshape(N)
    waf, wbf = wa.reshape(N), wb.reshape(N)

    xg, pos1, pos2 = _make_dispatch()(i1f, i2f, cnt.reshape(-1), xf)
    yd = _ffn(meta.reshape(NMETA), xg, W1, b1r, W2, b2r)
    out = _make_combine()(yd, pos1, pos2, waf, wbf)
    return out.reshape(B, L, Hd), aux.reshape(())


# speedup vs baseline: 1.2606x; 1.0058x over previous
import functools

import jax
import jax.numpy as jnp
from jax import lax
from jax.experimental import pallas as pl
from jax.experimental.pallas import tpu as pltpu
from jax.experimental.pallas import tpu_sc as plsc

H, E, FF = 1024, 8, 2048
N = 2048
W = 32
TPW = N // W
G = 8
BT = 256
P = 8192
NBLK = P // BT
STG = 192
NMETA = 64


def _router_kernel(x_ref, wr_ref, i1_ref, i2_ref, wa_ref, wb_ref, cnt_ref,
                   meta_ref, aux_ref):
    lg = jnp.dot(x_ref[...], wr_ref[...], preferred_element_type=jnp.float32)
    ids = lax.broadcasted_iota(jnp.int32, (N, E), 1)
    m1 = jnp.max(lg, axis=1, keepdims=True)
    i1 = jnp.min(jnp.where(lg == m1, ids, E), axis=1, keepdims=True)
    masked = jnp.where(ids == i1, -jnp.inf, lg)
    m2 = jnp.max(masked, axis=1, keepdims=True)
    i2 = jnp.min(jnp.where(masked == m2, ids, E), axis=1, keepdims=True)
    r = jnp.exp(m2 - m1)
    wa = 1.0 / (1.0 + r)
    i1_ref[...] = i1
    i2_ref[...] = i2
    wa_ref[...] = wa
    wb_ref[...] = 1.0 - wa
    ids16 = lax.broadcasted_iota(jnp.int32, (N, 16), 1)
    oh = (ids16 == i1).astype(jnp.float32) + (ids16 == i2).astype(jnp.float32)
    grp = lax.broadcasted_iota(jnp.int32, (W, N), 0)
    tokg = lax.broadcasted_iota(jnp.int32, (W, N), 1) // TPW
    sel = (grp == tokg).astype(jnp.float32)
    cnt = jnp.dot(sel, oh, preferred_element_type=jnp.float32)
    cnti = cnt.astype(jnp.int32)
    cnt_ref[...] = cnti
    rpad = (cnti + (G - 1)) & (-G)
    ptot = jnp.sum(rpad, axis=0, keepdims=True)
    region = (ptot + (BT - 1)) & (-BT)
    ii = lax.broadcasted_iota(jnp.int32, (16, 16), 0)
    jj = lax.broadcasted_iota(jnp.int32, (16, 16), 1)
    lt = (ii < jj).astype(jnp.float32)
    bend = (jnp.dot(region.astype(jnp.float32), lt,
                    preferred_element_type=jnp.float32)
            + region.astype(jnp.float32))
    bids = lax.broadcasted_iota(jnp.int32, (NMETA, 16), 0) * BT
    over = (bids.astype(jnp.float32) >= bend).astype(jnp.float32)
    colmask = lax.broadcasted_iota(jnp.int32, (NMETA, 16), 1) < E
    meta = jnp.sum(jnp.where(colmask, over, 0.0), axis=1, keepdims=True)
    meta_ref[...] = meta.astype(jnp.int32)
    counts = jnp.sum(oh[:, :E], axis=0, keepdims=True)
    load = counts / jnp.sum(counts)
    aux_ref[...] = 0.01 * jnp.sum(load * jnp.log(load + 1e-9), axis=1, keepdims=True)


def _router(xf, Wr):
    return pl.pallas_call(
        _router_kernel,
        grid=(1,),
        in_specs=[
            pl.BlockSpec((N, H), lambda i: (0, 0)),
            pl.BlockSpec((H, E), lambda i: (0, 0)),
        ],
        out_specs=[
            pl.BlockSpec((N, 1), lambda i: (0, 0)),
            pl.BlockSpec((N, 1), lambda i: (0, 0)),
            pl.BlockSpec((N, 1), lambda i: (0, 0)),
            pl.BlockSpec((N, 1), lambda i: (0, 0)),
            pl.BlockSpec((W, 16), lambda i: (0, 0)),
            pl.BlockSpec((NMETA, 1), lambda i: (0, 0)),
            pl.BlockSpec((1, 1), lambda i: (0, 0)),
        ],
        out_shape=[
            jax.ShapeDtypeStruct((N, 1), jnp.int32),
            jax.ShapeDtypeStruct((N, 1), jnp.int32),
            jax.ShapeDtypeStruct((N, 1), jnp.float32),
            jax.ShapeDtypeStruct((N, 1), jnp.float32),
            jax.ShapeDtypeStruct((W, 16), jnp.int32),
            jax.ShapeDtypeStruct((NMETA, 1), jnp.int32),
            jax.ShapeDtypeStruct((1, 1), jnp.float32),
        ],
    )(xf, Wr)


def _make_dispatch():
    mesh = plsc.VectorSubcoreMesh(core_axis_name="c", subcore_axis_name="s")

    @functools.partial(
        pl.kernel,
        mesh=mesh,
        out_type=[
            jax.ShapeDtypeStruct((P, H), jnp.float32),
            jax.ShapeDtypeStruct((N,), jnp.int32),
            jax.ShapeDtypeStruct((N,), jnp.int32),
        ],
        scratch_types=[
            pltpu.VMEM((TPW,), jnp.int32),
            pltpu.VMEM((TPW,), jnp.int32),
            pltpu.VMEM((W * 16,), jnp.int32),
            pltpu.VMEM((TPW,), jnp.int32),
            pltpu.VMEM((TPW,), jnp.int32),
            pltpu.VMEM((TPW, H), jnp.float32),
            pltpu.SemaphoreType.DMA,
            pltpu.SemaphoreType.DMA,
        ],
    )
    def dispatch(i1_hbm, i2_hbm, cnt_hbm, x_hbm,
                 xg_hbm, pos1_hbm, pos2_hbm,
                 i1_v, i2_v, cntg_v, pos1_v, pos2_v, xrows_v, sem, sem2):
        wid = lax.axis_index("s") * 2 + lax.axis_index("c")
        lane = lax.broadcasted_iota(jnp.int32, (16,), 0)
        t0 = pl.multiple_of(wid * TPW, TPW)
        cx = pltpu.async_copy(x_hbm.at[pl.ds(t0, TPW)], xrows_v, sem2)
        ci1 = pltpu.async_copy(i1_hbm.at[pl.ds(t0, TPW)], i1_v, sem2)
        ci2 = pltpu.async_copy(i2_hbm.at[pl.ds(t0, TPW)], i2_v, sem2)
        pltpu.sync_copy(cnt_hbm, cntg_v)

        zeros16 = jnp.zeros((16,), jnp.int32)
        totpad = zeros16
        mypre = zeros16
        myrow = zeros16
        for w in range(W):
            row = cntg_v[pl.ds(w * 16, 16)]
            rpad = (row + (G - 1)) & (-G)
            totpad = totpad + rpad
            mypre = mypre + rpad * (w < wid).astype(jnp.int32)
            myrow = myrow + row * (w == wid).astype(jnp.int32)
        region = (totpad + (BT - 1)) & (-BT)
        mypad = (myrow + (G - 1)) & (-G)

        starts = []
        bacc = jnp.int32(0)
        for e in range(E):
            starts.append(bacc + mypre[e])
            bacc = bacc + region[e]

        ks = [jnp.int32(0)] * E

        def assign(v, ks):
            rank = jnp.zeros((16,), jnp.int32)
            hist = jnp.zeros((16,), jnp.int32)
            for j in range(16):
                vj = v[j]
                rank = rank + jnp.where((v == vj) & (lane > j), 1, 0)
                hist = hist + jnp.where(lane == vj, 1, 0)
            slots = jnp.zeros((16,), jnp.int32)
            nks = []
            for e in range(E):
                m = v == e
                slots = jnp.where(m, starts[e] + ks[e] + rank, slots)
                nks.append(ks[e] + hist[e])
            return slots, nks

        cx.wait()
        ci1.wait()
        ci2.wait()
        cps = []
        for j in range(TPW // 16):
            v = i1_v[pl.ds(j * 16, 16)]
            slots, ks = assign(v, ks)
            pos1_v[pl.ds(j * 16, 16)] = slots
            cps.append(pltpu.async_copy(
                xrows_v.at[pl.ds(j * 16, 16)], xg_hbm.at[slots], sem))
        for j in range(TPW // 16):
            v = i2_v[pl.ds(j * 16, 16)]
            slots, ks = assign(v, ks)
            pos2_v[pl.ds(j * 16, 16)] = slots
            cps.append(pltpu.async_copy(
                xrows_v.at[pl.ds(j * 16, 16)], xg_hbm.at[slots], sem))
        for c in cps:
            c.wait()

        pltpu.sync_copy(pos1_v, pos1_hbm.at[pl.ds(t0, TPW)])
        pltpu.sync_copy(pos2_v, pos2_hbm.at[pl.ds(t0, TPW)])

    return dispatch


def _ffn_kernel(meta_ref, xg_ref, w1_ref, b1_ref, w2_ref, b2_ref, yd_ref):
    b = pl.program_id(0)
    e = meta_ref[b]

    @pl.when(e < E)
    def _():
        h = jnp.dot(xg_ref[...], w1_ref[0], preferred_element_type=jnp.float32)
        h = h + b1_ref[0]
        h = 0.5 * h * (1.0 + lax.erf(h * 0.7071067811865476))
        y = jnp.dot(h, w2_ref[0], preferred_element_type=jnp.float32) + b2_ref[0]
        yd_ref[...] = y


def _ffn(meta, xg, W1b, b1r, W2b, b2r):
    def clamp(m, b):
        return jnp.where(m[b] < E, m[b], 0)

    grid_spec = pltpu.PrefetchScalarGridSpec(
        num_scalar_prefetch=1,
        grid=(NBLK,),
        in_specs=[
            pl.BlockSpec((BT, H), lambda b, m: (b, 0)),
            pl.BlockSpec((1, H, FF), lambda b, m: (clamp(m, b), 0, 0)),
            pl.BlockSpec((1, 1, FF), lambda b, m: (clamp(m, b), 0, 0)),
            pl.BlockSpec((1, FF, H), lambda b, m: (clamp(m, b), 0, 0)),
            pl.BlockSpec((1, 1, H), lambda b, m: (clamp(m, b), 0, 0)),
        ],
        out_specs=pl.BlockSpec((BT, H), lambda b, m: (b, 0)),
    )
    return pl.pallas_call(
        _ffn_kernel,
        grid_spec=grid_spec,
        out_shape=jax.ShapeDtypeStruct((P, H), jnp.float32),
    )(meta, xg, W1b, b1r, W2b, b2r)


def _make_combine():
    mesh = plsc.VectorSubcoreMesh(core_axis_name="c", subcore_axis_name="s")
    HALF = 32

    @functools.partial(
        pl.kernel,
        mesh=mesh,
        out_type=jax.ShapeDtypeStruct((N, H), jnp.float32),
        scratch_types=[
            pltpu.VMEM((TPW,), jnp.int32),
            pltpu.VMEM((TPW,), jnp.int32),
            pltpu.VMEM((TPW,), jnp.float32),
            pltpu.VMEM((TPW,), jnp.float32),
            pltpu.VMEM((HALF, H), jnp.float32),
            pltpu.VMEM((HALF, H), jnp.float32),
            pltpu.SemaphoreType.DMA,
            pltpu.SemaphoreType.DMA,
        ],
    )
    def combine(yd_hbm, pos1_hbm, pos2_hbm, wa_hbm, wb_hbm, out_hbm,
                p1_v, p2_v, wa_v, wb_v, g1_v, g2_v, sem1, sem2):
        wid = lax.axis_index("s") * 2 + lax.axis_index("c")
        t0 = pl.multiple_of(wid * TPW, TPW)
        cw1 = pltpu.async_copy(wa_hbm.at[pl.ds(t0, TPW)], wa_v, sem1)
        cw2 = pltpu.async_copy(wb_hbm.at[pl.ds(t0, TPW)], wb_v, sem2)
        pltpu.sync_copy(pos1_hbm.at[pl.ds(t0, TPW)], p1_v)
        pltpu.sync_copy(pos2_hbm.at[pl.ds(t0, TPW)], p2_v)
        cw1.wait()
        cw2.wait()
        for hh in range(TPW // HALF):
            c1 = pltpu.async_copy(
                yd_hbm.at[p1_v.at[pl.ds(hh * HALF, HALF)]], g1_v, sem1)
            c2 = pltpu.async_copy(
                yd_hbm.at[p2_v.at[pl.ds(hh * HALF, HALF)]], g2_v, sem2)
            c1.wait()
            c2.wait()
            for t in range(HALF):
                tok = hh * HALF + t
                wav = wa_v[pl.ds((tok // 16) * 16, 16)]
                wbv = wb_v[pl.ds((tok // 16) * 16, 16)]
                a = wav[tok % 16]
                bsc = wbv[tok % 16]

                def body(i, _, t=t, a=a, bsc=bsc):
                    r1 = g1_v[t, pl.ds(i * 16, 16)]
                    r2 = g2_v[t, pl.ds(i * 16, 16)]
                    g1_v[t, pl.ds(i * 16, 16)] = a * r1 + bsc * r2
                    return 0

                lax.fori_loop(0, H // 16, body, 0)
            pltpu.sync_copy(g1_v, out_hbm.at[pl.ds(t0 + hh * HALF, HALF)])

    return combine


@jax.jit
def kernel(x, Wr, W1, b1, W2, b2):
    B, L, Hd = x.shape
    xf = x.reshape(-1, Hd)
    b1r = b1.reshape(E, 1, FF)
    b2r = b2.reshape(E, 1, H)

    i1, i2, wa, wb, cnt, meta, aux = _router(xf, Wr)
    i1f, i2f = i1.reshape(N), i2.reshape(N)
    waf, wbf = wa.reshape(N), wb.reshape(N)

    xg, pos1, pos2 = _make_dispatch()(i1f, i2f, cnt.reshape(-1), xf)
    yd = _ffn(meta.reshape(NMETA), xg, W1, b1r, W2, b2r)
    out = _make_combine()(yd, pos1, pos2, waf, wbf)
    return out.reshape(B, L, Hd), aux.reshape(())


# speedup vs baseline: 1.3028x; 1.0335x over previous
import functools

import jax
import jax.numpy as jnp
from jax import lax
from jax.experimental import pallas as pl
from jax.experimental.pallas import tpu as pltpu
from jax.experimental.pallas import tpu_sc as plsc

H, E, FF = 1024, 8, 2048
N = 2048
W = 32
TPW = N // W
G = 1
BT = 256
P = 8192
NBLK = P // BT
STG = 192
NMETA = 64


def _router_kernel(x_ref, wr_ref, i1_ref, i2_ref, wa_ref, wb_ref, cnt_ref,
                   meta_ref, aux_ref):
    lg = jnp.dot(x_ref[...], wr_ref[...], preferred_element_type=jnp.float32)
    ids = lax.broadcasted_iota(jnp.int32, (N, E), 1)
    m1 = jnp.max(lg, axis=1, keepdims=True)
    i1 = jnp.min(jnp.where(lg == m1, ids, E), axis=1, keepdims=True)
    masked = jnp.where(ids == i1, -jnp.inf, lg)
    m2 = jnp.max(masked, axis=1, keepdims=True)
    i2 = jnp.min(jnp.where(masked == m2, ids, E), axis=1, keepdims=True)
    r = jnp.exp(m2 - m1)
    wa = 1.0 / (1.0 + r)
    i1_ref[...] = i1
    i2_ref[...] = i2
    wa_ref[...] = wa
    wb_ref[...] = 1.0 - wa
    ids16 = lax.broadcasted_iota(jnp.int32, (N, 16), 1)
    oh = (ids16 == i1).astype(jnp.float32) + (ids16 == i2).astype(jnp.float32)
    grp = lax.broadcasted_iota(jnp.int32, (W, N), 0)
    tokg = lax.broadcasted_iota(jnp.int32, (W, N), 1) // TPW
    sel = (grp == tokg).astype(jnp.float32)
    cnt = jnp.dot(sel, oh, preferred_element_type=jnp.float32)
    cnti = cnt.astype(jnp.int32)
    cnt_ref[...] = cnti
    rpad = (cnti + (G - 1)) & (-G)
    ptot = jnp.sum(rpad, axis=0, keepdims=True)
    region = (ptot + (BT - 1)) & (-BT)
    ii = lax.broadcasted_iota(jnp.int32, (16, 16), 0)
    jj = lax.broadcasted_iota(jnp.int32, (16, 16), 1)
    lt = (ii < jj).astype(jnp.float32)
    bend = (jnp.dot(region.astype(jnp.float32), lt,
                    preferred_element_type=jnp.float32)
            + region.astype(jnp.float32))
    bids = lax.broadcasted_iota(jnp.int32, (NMETA, 16), 0) * BT
    over = (bids.astype(jnp.float32) >= bend).astype(jnp.float32)
    colmask = lax.broadcasted_iota(jnp.int32, (NMETA, 16), 1) < E
    meta = jnp.sum(jnp.where(colmask, over, 0.0), axis=1, keepdims=True)
    meta_ref[...] = meta.astype(jnp.int32)
    counts = jnp.sum(oh[:, :E], axis=0, keepdims=True)
    load = counts / jnp.sum(counts)
    aux_ref[...] = 0.01 * jnp.sum(load * jnp.log(load + 1e-9), axis=1, keepdims=True)


def _router(xf, Wr):
    return pl.pallas_call(
        _router_kernel,
        grid=(1,),
        in_specs=[
            pl.BlockSpec((N, H), lambda i: (0, 0)),
            pl.BlockSpec((H, E), lambda i: (0, 0)),
        ],
        out_specs=[
            pl.BlockSpec((N, 1), lambda i: (0, 0)),
            pl.BlockSpec((N, 1), lambda i: (0, 0)),
            pl.BlockSpec((N, 1), lambda i: (0, 0)),
            pl.BlockSpec((N, 1), lambda i: (0, 0)),
            pl.BlockSpec((W, 16), lambda i: (0, 0)),
            pl.BlockSpec((NMETA, 1), lambda i: (0, 0)),
            pl.BlockSpec((1, 1), lambda i: (0, 0)),
        ],
        out_shape=[
            jax.ShapeDtypeStruct((N, 1), jnp.int32),
            jax.ShapeDtypeStruct((N, 1), jnp.int32),
            jax.ShapeDtypeStruct((N, 1), jnp.float32),
            jax.ShapeDtypeStruct((N, 1), jnp.float32),
            jax.ShapeDtypeStruct((W, 16), jnp.int32),
            jax.ShapeDtypeStruct((NMETA, 1), jnp.int32),
            jax.ShapeDtypeStruct((1, 1), jnp.float32),
        ],
    )(xf, Wr)


def _make_dispatch():
    mesh = plsc.VectorSubcoreMesh(core_axis_name="c", subcore_axis_name="s")

    @functools.partial(
        pl.kernel,
        mesh=mesh,
        out_type=[
            jax.ShapeDtypeStruct((P, H), jnp.float32),
            jax.ShapeDtypeStruct((N,), jnp.int32),
            jax.ShapeDtypeStruct((N,), jnp.int32),
        ],
        scratch_types=[
            pltpu.VMEM((TPW,), jnp.int32),
            pltpu.VMEM((TPW,), jnp.int32),
            pltpu.VMEM((W * 16,), jnp.int32),
            pltpu.VMEM((TPW,), jnp.int32),
            pltpu.VMEM((TPW,), jnp.int32),
            pltpu.VMEM((TPW, H), jnp.float32),
            pltpu.SemaphoreType.DMA,
            pltpu.SemaphoreType.DMA,
        ],
    )
    def dispatch(i1_hbm, i2_hbm, cnt_hbm, x_hbm,
                 xg_hbm, pos1_hbm, pos2_hbm,
                 i1_v, i2_v, cntg_v, pos1_v, pos2_v, xrows_v, sem, sem2):
        wid = lax.axis_index("s") * 2 + lax.axis_index("c")
        lane = lax.broadcasted_iota(jnp.int32, (16,), 0)
        t0 = pl.multiple_of(wid * TPW, TPW)
        cx = pltpu.async_copy(x_hbm.at[pl.ds(t0, TPW)], xrows_v, sem2)
        ci1 = pltpu.async_copy(i1_hbm.at[pl.ds(t0, TPW)], i1_v, sem2)
        ci2 = pltpu.async_copy(i2_hbm.at[pl.ds(t0, TPW)], i2_v, sem2)
        pltpu.sync_copy(cnt_hbm, cntg_v)

        zeros16 = jnp.zeros((16,), jnp.int32)
        totpad = zeros16
        mypre = zeros16
        myrow = zeros16
        for w in range(W):
            row = cntg_v[pl.ds(w * 16, 16)]
            rpad = (row + (G - 1)) & (-G)
            totpad = totpad + rpad
            mypre = mypre + rpad * (w < wid).astype(jnp.int32)
            myrow = myrow + row * (w == wid).astype(jnp.int32)
        region = (totpad + (BT - 1)) & (-BT)
        mypad = (myrow + (G - 1)) & (-G)

        starts = []
        bacc = jnp.int32(0)
        for e in range(E):
            starts.append(bacc + mypre[e])
            bacc = bacc + region[e]

        ks = [jnp.int32(0)] * E

        def assign(v, ks):
            rank = jnp.zeros((16,), jnp.int32)
            hist = jnp.zeros((16,), jnp.int32)
            for j in range(16):
                vj = v[j]
                rank = rank + jnp.where((v == vj) & (lane > j), 1, 0)
                hist = hist + jnp.where(lane == vj, 1, 0)
            slots = jnp.zeros((16,), jnp.int32)
            nks = []
            for e in range(E):
                m = v == e
                slots = jnp.where(m, starts[e] + ks[e] + rank, slots)
                nks.append(ks[e] + hist[e])
            return slots, nks

        cx.wait()
        ci1.wait()
        ci2.wait()
        cps = []
        for j in range(TPW // 16):
            v = i1_v[pl.ds(j * 16, 16)]
            slots, ks = assign(v, ks)
            pos1_v[pl.ds(j * 16, 16)] = slots
            cps.append(pltpu.async_copy(
                xrows_v.at[pl.ds(j * 16, 16)], xg_hbm.at[slots], sem))
        for j in range(TPW // 16):
            v = i2_v[pl.ds(j * 16, 16)]
            slots, ks = assign(v, ks)
            pos2_v[pl.ds(j * 16, 16)] = slots
            cps.append(pltpu.async_copy(
                xrows_v.at[pl.ds(j * 16, 16)], xg_hbm.at[slots], sem))
        for c in cps:
            c.wait()

        pltpu.sync_copy(pos1_v, pos1_hbm.at[pl.ds(t0, TPW)])
        pltpu.sync_copy(pos2_v, pos2_hbm.at[pl.ds(t0, TPW)])

    return dispatch


def _ffn_kernel(meta_ref, xg_ref, w1_ref, b1_ref, w2_ref, b2_ref, yd_ref):
    b = pl.program_id(0)
    e = meta_ref[b]

    @pl.when(e < E)
    def _():
        h = jnp.dot(xg_ref[...], w1_ref[0], preferred_element_type=jnp.float32)
        h = h + b1_ref[0]
        h = 0.5 * h * (1.0 + lax.erf(h * 0.7071067811865476))
        y = jnp.dot(h, w2_ref[0], preferred_element_type=jnp.float32) + b2_ref[0]
        yd_ref[...] = y


def _ffn(meta, xg, W1b, b1r, W2b, b2r):
    def clamp(m, b):
        return jnp.where(m[b] < E, m[b], 0)

    grid_spec = pltpu.PrefetchScalarGridSpec(
        num_scalar_prefetch=1,
        grid=(NBLK,),
        in_specs=[
            pl.BlockSpec((BT, H), lambda b, m: (b, 0)),
            pl.BlockSpec((1, H, FF), lambda b, m: (clamp(m, b), 0, 0)),
            pl.BlockSpec((1, 1, FF), lambda b, m: (clamp(m, b), 0, 0)),
            pl.BlockSpec((1, FF, H), lambda b, m: (clamp(m, b), 0, 0)),
            pl.BlockSpec((1, 1, H), lambda b, m: (clamp(m, b), 0, 0)),
        ],
        out_specs=pl.BlockSpec((BT, H), lambda b, m: (b, 0)),
    )
    return pl.pallas_call(
        _ffn_kernel,
        grid_spec=grid_spec,
        out_shape=jax.ShapeDtypeStruct((P, H), jnp.float32),
    )(meta, xg, W1b, b1r, W2b, b2r)


def _make_combine():
    mesh = plsc.VectorSubcoreMesh(core_axis_name="c", subcore_axis_name="s")
    HALF = 32

    @functools.partial(
        pl.kernel,
        mesh=mesh,
        out_type=jax.ShapeDtypeStruct((N, H), jnp.float32),
        scratch_types=[
            pltpu.VMEM((TPW,), jnp.int32),
            pltpu.VMEM((TPW,), jnp.int32),
            pltpu.VMEM((TPW,), jnp.float32),
            pltpu.VMEM((TPW,), jnp.float32),
            pltpu.VMEM((HALF, H), jnp.float32),
            pltpu.VMEM((HALF, H), jnp.float32),
            pltpu.SemaphoreType.DMA,
            pltpu.SemaphoreType.DMA,
        ],
    )
    def combine(yd_hbm, pos1_hbm, pos2_hbm, wa_hbm, wb_hbm, out_hbm,
                p1_v, p2_v, wa_v, wb_v, g1_v, g2_v, sem1, sem2):
        wid = lax.axis_index("s") * 2 + lax.axis_index("c")
        t0 = pl.multiple_of(wid * TPW, TPW)
        cw1 = pltpu.async_copy(wa_hbm.at[pl.ds(t0, TPW)], wa_v, sem1)
        cw2 = pltpu.async_copy(wb_hbm.at[pl.ds(t0, TPW)], wb_v, sem2)
        pltpu.sync_copy(pos1_hbm.at[pl.ds(t0, TPW)], p1_v)
        pltpu.sync_copy(pos2_hbm.at[pl.ds(t0, TPW)], p2_v)
        cw1.wait()
        cw2.wait()
        for hh in range(TPW // HALF):
            c1 = pltpu.async_copy(
                yd_hbm.at[p1_v.at[pl.ds(hh * HALF, HALF)]], g1_v, sem1)
            c2 = pltpu.async_copy(
                yd_hbm.at[p2_v.at[pl.ds(hh * HALF, HALF)]], g2_v, sem2)
            c1.wait()
            c2.wait()
            for t in range(HALF):
                tok = hh * HALF + t
                wav = wa_v[pl.ds((tok // 16) * 16, 16)]
                wbv = wb_v[pl.ds((tok // 16) * 16, 16)]
                a = wav[tok % 16]
                bsc = wbv[tok % 16]

                def body(i, _, t=t, a=a, bsc=bsc):
                    r1 = g1_v[t, pl.ds(i * 16, 16)]
                    r2 = g2_v[t, pl.ds(i * 16, 16)]
                    g1_v[t, pl.ds(i * 16, 16)] = a * r1 + bsc * r2
                    return 0

                lax.fori_loop(0, H // 16, body, 0)
            pltpu.sync_copy(g1_v, out_hbm.at[pl.ds(t0 + hh * HALF, HALF)])

    return combine


@jax.jit
def kernel(x, Wr, W1, b1, W2, b2):
    B, L, Hd = x.shape
    xf = x.reshape(-1, Hd)
    b1r = b1.reshape(E, 1, FF)
    b2r = b2.reshape(E, 1, H)

    i1, i2, wa, wb, cnt, meta, aux = _router(xf, Wr)
    i1f, i2f = i1.reshape(N), i2.reshape(N)
    waf, wbf = wa.reshape(N), wb.reshape(N)

    xg, pos1, pos2 = _make_dispatch()(i1f, i2f, cnt.reshape(-1), xf)
    yd = _ffn(meta.reshape(NMETA), xg, W1, b1r, W2, b2r)
    out = _make_combine()(yd, pos1, pos2, waf, wbf)
    return out.reshape(B, L, Hd), aux.reshape(())


# speedup vs baseline: 1.3028x; 1.0000x over previous
import functools

import jax
import jax.numpy as jnp
from jax import lax
from jax.experimental import pallas as pl
from jax.experimental.pallas import tpu as pltpu
from jax.experimental.pallas import tpu_sc as plsc

H, E, FF = 1024, 8, 2048
N = 2048
W = 32
TPW = N // W
G = 1
BT = 256
P = 8192
NBLK = P // BT
NMETA = 64


def _router_kernel(x_ref, wr_ref, i1_ref, i2_ref, wa_ref, wb_ref, cnt_ref,
                   meta_ref, aux_ref):
    lg = jnp.dot(x_ref[...], wr_ref[...], preferred_element_type=jnp.float32)
    ids = lax.broadcasted_iota(jnp.int32, (N, E), 1)
    m1 = jnp.max(lg, axis=1, keepdims=True)
    i1 = jnp.min(jnp.where(lg == m1, ids, E), axis=1, keepdims=True)
    masked = jnp.where(ids == i1, -jnp.inf, lg)
    m2 = jnp.max(masked, axis=1, keepdims=True)
    i2 = jnp.min(jnp.where(masked == m2, ids, E), axis=1, keepdims=True)
    r = jnp.exp(m2 - m1)
    wa = 1.0 / (1.0 + r)
    i1_ref[...] = i1
    i2_ref[...] = i2
    wa_ref[...] = wa
    wb_ref[...] = 1.0 - wa
    ids16 = lax.broadcasted_iota(jnp.int32, (N, 16), 1)
    oh = (ids16 == i1).astype(jnp.float32) + (ids16 == i2).astype(jnp.float32)
    grp = lax.broadcasted_iota(jnp.int32, (W, N), 0)
    tokg = lax.broadcasted_iota(jnp.int32, (W, N), 1) // TPW
    sel = (grp == tokg).astype(jnp.float32)
    cnt = jnp.dot(sel, oh, preferred_element_type=jnp.float32)
    cnti = cnt.astype(jnp.int32)
    cnt_ref[...] = cnti
    rpad = (cnti + (G - 1)) & (-G)
    ptot = jnp.sum(rpad, axis=0, keepdims=True)
    region = (ptot + (BT - 1)) & (-BT)
    ii = lax.broadcasted_iota(jnp.int32, (16, 16), 0)
    jj = lax.broadcasted_iota(jnp.int32, (16, 16), 1)
    lt = (ii < jj).astype(jnp.float32)
    bend = (jnp.dot(region.astype(jnp.float32), lt,
                    preferred_element_type=jnp.float32)
            + region.astype(jnp.float32))
    bids = lax.broadcasted_iota(jnp.int32, (NMETA, 16), 0) * BT
    over = (bids.astype(jnp.float32) >= bend).astype(jnp.float32)
    colmask = lax.broadcasted_iota(jnp.int32, (NMETA, 16), 1) < E
    meta = jnp.sum(jnp.where(colmask, over, 0.0), axis=1, keepdims=True)
    meta_ref[...] = meta.astype(jnp.int32)
    counts = jnp.sum(oh[:, :E], axis=0, keepdims=True)
    load = counts / jnp.sum(counts)
    aux_ref[...] = 0.01 * jnp.sum(load * jnp.log(load + 1e-9), axis=1, keepdims=True)


def _router(xf, Wr):
    return pl.pallas_call(
        _router_kernel,
        grid=(1,),
        in_specs=[
            pl.BlockSpec((N, H), lambda i: (0, 0)),
            pl.BlockSpec((H, E), lambda i: (0, 0)),
        ],
        out_specs=[
            pl.BlockSpec((N, 1), lambda i: (0, 0)),
            pl.BlockSpec((N, 1), lambda i: (0, 0)),
            pl.BlockSpec((N, 1), lambda i: (0, 0)),
            pl.BlockSpec((N, 1), lambda i: (0, 0)),
            pl.BlockSpec((W, 16), lambda i: (0, 0)),
            pl.BlockSpec((NMETA, 1), lambda i: (0, 0)),
            pl.BlockSpec((1, 1), lambda i: (0, 0)),
        ],
        out_shape=[
            jax.ShapeDtypeStruct((N, 1), jnp.int32),
            jax.ShapeDtypeStruct((N, 1), jnp.int32),
            jax.ShapeDtypeStruct((N, 1), jnp.float32),
            jax.ShapeDtypeStruct((N, 1), jnp.float32),
            jax.ShapeDtypeStruct((W, 16), jnp.int32),
            jax.ShapeDtypeStruct((NMETA, 1), jnp.int32),
            jax.ShapeDtypeStruct((1, 1), jnp.float32),
        ],
    )(xf, Wr)


def _make_dispatch():
    mesh = plsc.VectorSubcoreMesh(core_axis_name="c", subcore_axis_name="s")

    @functools.partial(
        pl.kernel,
        mesh=mesh,
        out_type=[
            jax.ShapeDtypeStruct((P, H), jnp.float32),
            jax.ShapeDtypeStruct((N,), jnp.int32),
            jax.ShapeDtypeStruct((N,), jnp.int32),
        ],
        scratch_types=[
            pltpu.VMEM((TPW,), jnp.int32),
            pltpu.VMEM((TPW,), jnp.int32),
            pltpu.VMEM((W * 16,), jnp.int32),
            pltpu.VMEM((TPW,), jnp.int32),
            pltpu.VMEM((TPW,), jnp.int32),
            pltpu.VMEM((TPW, H), jnp.float32),
            pltpu.SemaphoreType.DMA,
            pltpu.SemaphoreType.DMA,
        ],
    )
    def dispatch(i1_hbm, i2_hbm, cnt_hbm, x_hbm,
                 xg_hbm, pos1_hbm, pos2_hbm,
                 i1_v, i2_v, cntg_v, pos1_v, pos2_v, xrows_v, sem, sem2):
        wid = lax.axis_index("s") * 2 + lax.axis_index("c")
        lane = lax.broadcasted_iota(jnp.int32, (16,), 0)
        t0 = pl.multiple_of(wid * TPW, TPW)
        cx = pltpu.async_copy(x_hbm.at[pl.ds(t0, TPW)], xrows_v, sem2)
        ci1 = pltpu.async_copy(i1_hbm.at[pl.ds(t0, TPW)], i1_v, sem2)
        ci2 = pltpu.async_copy(i2_hbm.at[pl.ds(t0, TPW)], i2_v, sem2)
        pltpu.sync_copy(cnt_hbm, cntg_v)

        zeros16 = jnp.zeros((16,), jnp.int32)
        totpad = zeros16
        mypre = zeros16
        myrow = zeros16
        for w in range(W):
            row = cntg_v[pl.ds(w * 16, 16)]
            rpad = (row + (G - 1)) & (-G)
            totpad = totpad + rpad
            mypre = mypre + rpad * (w < wid).astype(jnp.int32)
            myrow = myrow + row * (w == wid).astype(jnp.int32)
        region = (totpad + (BT - 1)) & (-BT)
        mypad = (myrow + (G - 1)) & (-G)

        starts = []
        bacc = jnp.int32(0)
        for e in range(E):
            starts.append(bacc + mypre[e])
            bacc = bacc + region[e]

        ks = [jnp.int32(0)] * E

        def assign(v, ks):
            rank = jnp.zeros((16,), jnp.int32)
            hist = jnp.zeros((16,), jnp.int32)
            for j in range(16):
                vj = v[j]
                rank = rank + jnp.where((v == vj) & (lane > j), 1, 0)
                hist = hist + jnp.where(lane == vj, 1, 0)
            slots = jnp.zeros((16,), jnp.int32)
            nks = []
            for e in range(E):
                m = v == e
                slots = jnp.where(m, starts[e] + ks[e] + rank, slots)
                nks.append(ks[e] + hist[e])
            return slots, nks

        cx.wait()
        ci1.wait()
        ci2.wait()
        cps = []
        for j in range(TPW // 16):
            v = i1_v[pl.ds(j * 16, 16)]
            slots, ks = assign(v, ks)
            pos1_v[pl.ds(j * 16, 16)] = slots
            cps.append(pltpu.async_copy(
                xrows_v.at[pl.ds(j * 16, 16)], xg_hbm.at[slots], sem))
        for j in range(TPW // 16):
            v = i2_v[pl.ds(j * 16, 16)]
            slots, ks = assign(v, ks)
            pos2_v[pl.ds(j * 16, 16)] = slots
            cps.append(pltpu.async_copy(
                xrows_v.at[pl.ds(j * 16, 16)], xg_hbm.at[slots], sem))
        for c in cps:
            c.wait()

        pltpu.sync_copy(pos1_v, pos1_hbm.at[pl.ds(t0, TPW)])
        pltpu.sync_copy(pos2_v, pos2_hbm.at[pl.ds(t0, TPW)])

    return dispatch


def _ffn_kernel(meta_ref, xg_ref, w1_ref, b1_ref, w2_ref, b2_ref, yd_ref):
    b = pl.program_id(0)
    e = meta_ref[b]

    @pl.when(e < E)
    def _():
        h = jnp.dot(xg_ref[...], w1_ref[0], preferred_element_type=jnp.float32)
        h = h + b1_ref[0]
        h = 0.5 * h * (1.0 + lax.erf(h * 0.7071067811865476))
        y = jnp.dot(h, w2_ref[0], preferred_element_type=jnp.float32) + b2_ref[0]
        yd_ref[...] = y


def _ffn(meta, xg, W1b, b1r, W2b, b2r):
    def clamp(m, b):
        return jnp.where(m[b] < E, m[b], 0)

    grid_spec = pltpu.PrefetchScalarGridSpec(
        num_scalar_prefetch=1,
        grid=(NBLK,),
        in_specs=[
            pl.BlockSpec((BT, H), lambda b, m: (b, 0)),
            pl.BlockSpec((1, H, FF), lambda b, m: (clamp(m, b), 0, 0)),
            pl.BlockSpec((1, 1, FF), lambda b, m: (clamp(m, b), 0, 0)),
            pl.BlockSpec((1, FF, H), lambda b, m: (clamp(m, b), 0, 0)),
            pl.BlockSpec((1, 1, H), lambda b, m: (clamp(m, b), 0, 0)),
        ],
        out_specs=pl.BlockSpec((BT, H), lambda b, m: (b, 0)),
    )
    return pl.pallas_call(
        _ffn_kernel,
        grid_spec=grid_spec,
        out_shape=jax.ShapeDtypeStruct((P, H), jnp.float32),
    )(meta, xg, W1b, b1r, W2b, b2r)


def _make_combine():
    mesh = plsc.VectorSubcoreMesh(core_axis_name="c", subcore_axis_name="s")
    HALF = 32

    @functools.partial(
        pl.kernel,
        mesh=mesh,
        out_type=jax.ShapeDtypeStruct((N, H), jnp.float32),
        scratch_types=[
            pltpu.VMEM((TPW,), jnp.int32),
            pltpu.VMEM((TPW,), jnp.int32),
            pltpu.VMEM((TPW,), jnp.float32),
            pltpu.VMEM((TPW,), jnp.float32),
            pltpu.VMEM((HALF, H), jnp.float32),
            pltpu.VMEM((HALF, H), jnp.float32),
            pltpu.SemaphoreType.DMA,
            pltpu.SemaphoreType.DMA,
        ],
    )
    def combine(yd_hbm, pos1_hbm, pos2_hbm, wa_hbm, wb_hbm, out_hbm,
                p1_v, p2_v, wa_v, wb_v, g1_v, g2_v, sem1, sem2):
        wid = lax.axis_index("s") * 2 + lax.axis_index("c")
        t0 = pl.multiple_of(wid * TPW, TPW)
        cw1 = pltpu.async_copy(wa_hbm.at[pl.ds(t0, TPW)], wa_v, sem1)
        cw2 = pltpu.async_copy(wb_hbm.at[pl.ds(t0, TPW)], wb_v, sem2)
        pltpu.sync_copy(pos1_hbm.at[pl.ds(t0, TPW)], p1_v)
        pltpu.sync_copy(pos2_hbm.at[pl.ds(t0, TPW)], p2_v)
        cw1.wait()
        cw2.wait()
        for hh in range(TPW // HALF):
            c1 = pltpu.async_copy(
                yd_hbm.at[p1_v.at[pl.ds(hh * HALF, HALF)]], g1_v, sem1)
            c2 = pltpu.async_copy(
                yd_hbm.at[p2_v.at[pl.ds(hh * HALF, HALF)]], g2_v, sem2)
            c1.wait()
            c2.wait()
            for t in range(HALF):
                tok = hh * HALF + t
                wav = wa_v[pl.ds((tok // 16) * 16, 16)]
                wbv = wb_v[pl.ds((tok // 16) * 16, 16)]
                a = wav[tok % 16]
                bsc = wbv[tok % 16]

                def body(i, _, t=t, a=a, bsc=bsc):
                    r1 = g1_v[t, pl.ds(i * 16, 16)]
                    r2 = g2_v[t, pl.ds(i * 16, 16)]
                    g1_v[t, pl.ds(i * 16, 16)] = a * r1 + bsc * r2
                    return 0

                lax.fori_loop(0, H // 16, body, 0)
            pltpu.sync_copy(g1_v, out_hbm.at[pl.ds(t0 + hh * HALF, HALF)])

    return combine


@jax.jit
def kernel(x, Wr, W1, b1, W2, b2):
    B, L, Hd = x.shape
    xf = x.reshape(-1, Hd)
    b1r = b1.reshape(E, 1, FF)
    b2r = b2.reshape(E, 1, H)

    i1, i2, wa, wb, cnt, meta, aux = _router(xf, Wr)
    i1f, i2f = i1.reshape(N), i2.reshape(N)
    waf, wbf = wa.reshape(N), wb.reshape(N)

    xg, pos1, pos2 = _make_dispatch()(i1f, i2f, cnt.reshape(-1), xf)
    yd = _ffn(meta.reshape(NMETA), xg, W1, b1r, W2, b2r)
    out = _make_combine()(yd, pos1, pos2, waf, wbf)
    return out.reshape(B, L, Hd), aux.reshape(())
